# baseline (speedup 1.0000x reference)
import functools

import jax
import jax.numpy as jnp
from jax import lax
from jax.experimental import pallas as pl
from jax.experimental.pallas import tpu as pltpu
from jax.experimental.pallas import tpu_sc as plsc

_N = 20000
_NS = 32768
_R, _C = 256, 128
_K = 1000
_KP = 1024
_KR = _KP // _C
_NMS_THRESH = 0.7
_MIN_SIZE = 0.001
_SCORE_THRESH = 0.0
_NEG_INF = float("-inf")


def _topk_kernel(score_ref, probs_ref, idx_ref):
    s = score_ref[0]
    r_io = jax.lax.broadcasted_iota(jnp.int32, (_R, _C), 0)
    c_io = jax.lax.broadcasted_iota(jnp.int32, (_R, _C), 1)
    idx = r_io * _C + c_io

    def partner(a, j):
        if j < _C:
            return jnp.where((c_io & j) == 0,
                             jnp.roll(a, -j, axis=1), jnp.roll(a, j, axis=1))
        jr = j // _C
        return jnp.where((r_io & jr) == 0,
                         jnp.roll(a, -jr, axis=0), jnp.roll(a, jr, axis=0))

    k = 2
    while k <= _NS:
        j = k // 2
        while j >= 1:
            ps = partner(s, j)
            pidx = partner(idx, j)
            self_better = (s > ps) | ((s == ps) & (idx < pidx))
            is_lo = ((c_io & j) == 0) if j < _C else ((r_io & (j // _C)) == 0)
            if k < _C:
                up = (c_io & k) == 0
            elif k < _NS:
                up = (r_io & (k // _C)) == 0
            else:
                up = None
            want_better = is_lo if up is None else (is_lo == up)
            take = self_better != want_better
            s = jnp.where(take, ps, s)
            idx = jnp.where(take, pidx, idx)
            j //= 2
        k *= 2

    probs_ref[0] = jax.nn.sigmoid(s[0:_KR, :])
    idx_ref[0] = idx[0:_KR, :] + pl.program_id(0) * _N


def _sc_gather(table_hbm, idx_hbm, out_hbm, idx_v, rows_v, sem):
    nw = 32
    bpw = (2 * _KP) // nw
    wid = lax.axis_index("s") * 2 + lax.axis_index("c")
    base = wid * bpw
    pltpu.sync_copy(idx_hbm.at[pl.ds(base, bpw)], idx_v)
    pltpu.async_copy(table_hbm.at[idx_v], rows_v, sem).wait()
    pltpu.sync_copy(rows_v, out_hbm.at[pl.ds(base, bpw)])


def _nms_kernel(brow_ref, bcol_ref, probs_ref, hw_ref, out_ref, m_ref):
    h11 = hw_ref[0:1, 0:1]
    w11 = hw_ref[0:1, 1:2]
    x1r = jnp.clip(brow_ref[0, 0:1, :], 0.0, w11)
    y1r = jnp.clip(brow_ref[0, 1:2, :], 0.0, h11)
    x2r = jnp.clip(brow_ref[0, 2:3, :], 0.0, w11)
    y2r = jnp.clip(brow_ref[0, 3:4, :], 0.0, h11)
    pr = probs_ref[0]
    ws = x2r - x1r
    hs = y2r - y1r
    area_r = ws * hs
    valid = ((ws >= _MIN_SIZE) & (hs >= _MIN_SIZE)
             & (pr >= _SCORE_THRESH)).astype(jnp.float32)

    chunk = 128
    for ch in range(_KP // chunk):
        colc = bcol_ref[0, ch * chunk:(ch + 1) * chunk, :]
        x1c = jnp.clip(colc[:, 0:1], 0.0, w11)
        y1c = jnp.clip(colc[:, 1:2], 0.0, h11)
        x2c = jnp.clip(colc[:, 2:3], 0.0, w11)
        y2c = jnp.clip(colc[:, 3:4], 0.0, h11)
        area_c = (x2c - x1c) * (y2c - y1c)
        xx1 = jnp.maximum(x1c, x1r)
        yy1 = jnp.maximum(y1c, y1r)
        xx2 = jnp.minimum(x2c, x2r)
        yy2 = jnp.minimum(y2c, y2r)
        inter = (jnp.clip(xx2 - xx1, 0.0, None)
                 * jnp.clip(yy2 - yy1, 0.0, None))
        union = area_c + area_r - inter
        iou = inter / jnp.maximum(union, 1e-9)
        jio = jax.lax.broadcasted_iota(jnp.int32, (chunk, _KP), 0) + ch * chunk
        iio = jax.lax.broadcasted_iota(jnp.int32, (chunk, _KP), 1)
        m_ref[ch * chunk:(ch + 1) * chunk, :] = jnp.where(
            (iou > _NMS_THRESH) & (jio < iio), 1.0, 0.0)

    def cond(carry):
        return carry[1]

    def body(carry):
        keep, _ = carry
        supp = jax.lax.dot_general(
            keep, m_ref[...], (((1,), (0,)), ((), ())),
            preferred_element_type=jnp.float32)
        new = jnp.where(supp > 0.0, 0.0, valid)
        return new, jnp.any(new != keep)

    keep, _ = jax.lax.while_loop(cond, body, (valid, jnp.bool_(True)))

    out_ref[0, 0:1, :] = x1r * keep
    out_ref[0, 1:2, :] = y1r * keep
    out_ref[0, 2:3, :] = x2r * keep
    out_ref[0, 3:4, :] = y2r * keep
    out_ref[0, 4:5, :] = pr * keep


def kernel(proposals, objectness, image_height, image_width):
    B = proposals.shape[0]
    f32 = jnp.float32

    scores = jnp.concatenate(
        [objectness.astype(f32), jnp.full((B, _NS - _N), _NEG_INF, f32)],
        axis=1).reshape(B, _R, _C)
    hw = jnp.stack([jnp.asarray(image_height, f32),
                    jnp.asarray(image_width, f32)]).reshape(1, 2)

    probs, idx = pl.pallas_call(
        _topk_kernel,
        grid=(B,),
        in_specs=[pl.BlockSpec((1, _R, _C), lambda b: (b, 0, 0))],
        out_specs=[pl.BlockSpec((1, _KR, _C), lambda b: (b, 0, 0)),
                   pl.BlockSpec((1, _KR, _C), lambda b: (b, 0, 0))],
        out_shape=[jax.ShapeDtypeStruct((B, _KR, _C), f32),
                   jax.ShapeDtypeStruct((B, _KR, _C), jnp.int32)],
    )(scores)

    probs_row = probs.reshape(B, 1, _KP)
    idx_flat = idx.reshape(B * _KP)
    table = jnp.concatenate(
        [proposals.astype(f32), jnp.zeros((B, _N, 12), f32)],
        axis=2).reshape(B * _N, 16)

    mesh = plsc.VectorSubcoreMesh(core_axis_name="c", subcore_axis_name="s")
    bpw = (2 * _KP) // 32
    gathered = functools.partial(
        pl.kernel,
        mesh=mesh,
        out_type=jax.ShapeDtypeStruct((B * _KP, 16), f32),
        scratch_types=[pltpu.VMEM((bpw,), jnp.int32),
                       pltpu.VMEM((bpw, 16), f32),
                       pltpu.SemaphoreType.DMA],
        compiler_params=pltpu.CompilerParams(use_tc_tiling_on_sc=False),
    )(_sc_gather)(table, idx_flat)

    bcol = gathered.reshape(B, _KP, 16)
    brow = jnp.transpose(bcol[:, :, 0:4], (0, 2, 1))

    out = pl.pallas_call(
        _nms_kernel,
        grid=(B,),
        in_specs=[
            pl.BlockSpec((1, 4, _KP), lambda b: (b, 0, 0)),
            pl.BlockSpec((1, _KP, 16), lambda b: (b, 0, 0)),
            pl.BlockSpec((1, 1, _KP), lambda b: (b, 0, 0)),
            pl.BlockSpec((1, 2), lambda b: (0, 0)),
        ],
        out_specs=pl.BlockSpec((1, 5, _KP), lambda b: (b, 0, 0)),
        out_shape=jax.ShapeDtypeStruct((B, 5, _KP), f32),
        scratch_shapes=[pltpu.VMEM((_KP, _KP), f32)],
    )(brow, bcol, probs_row, hw)

    return jnp.transpose(out, (0, 2, 1))[:, :_K, :]


# speedup vs baseline: 1.5211x; 1.5211x over previous
import jax
import jax.numpy as jnp
from jax.experimental import pallas as pl
from jax.experimental.pallas import tpu as pltpu

_N = 20000
_NS = 32768
_R, _C = 256, 128
_K = 1000
_KP = 1024
_KR = _KP // _C
_NMS_THRESH = 0.7
_MIN_SIZE = 0.001
_SCORE_THRESH = 0.0
_NEG_INF = float("-inf")


def _topk_kernel(score_ref, boxes_ref, hw_ref, out_ref):
    s = score_ref[0]
    arrs = [s,
            (jax.lax.broadcasted_iota(jnp.int32, (_R, _C), 0) * _C
             + jax.lax.broadcasted_iota(jnp.int32, (_R, _C), 1)),
            boxes_ref[0, 0], boxes_ref[0, 1], boxes_ref[0, 2], boxes_ref[0, 3]]

    def iotas(w):
        return (jax.lax.broadcasted_iota(jnp.int32, (w, _C), 0),
                jax.lax.broadcasted_iota(jnp.int32, (w, _C), 1))

    def partner(a, j, r_io, c_io):
        if j < _C:
            return jnp.where((c_io & j) == 0,
                             jnp.roll(a, -j, axis=1), jnp.roll(a, j, axis=1))
        jr = j // _C
        return jnp.where((r_io & jr) == 0,
                         jnp.roll(a, -jr, axis=0), jnp.roll(a, jr, axis=0))

    def exchange(arrs, j, want_better, r_io, c_io):
        ps = partner(arrs[0], j, r_io, c_io)
        pidx = partner(arrs[1], j, r_io, c_io)
        self_better = (arrs[0] > ps) | ((arrs[0] == ps) & (arrs[1] < pidx))
        take = self_better != want_better
        out = [jnp.where(take, ps, arrs[0]), jnp.where(take, pidx, arrs[1])]
        for a in arrs[2:]:
            out.append(jnp.where(take, partner(a, j, r_io, c_io), a))
        return out

    r_io, c_io = iotas(_R)
    k = 2
    while k <= _KP:
        j = k // 2
        while j >= 1:
            is_lo = ((c_io & j) == 0) if j < _C else ((r_io & (j // _C)) == 0)
            up = ((c_io & k) == 0) if k < _C else ((r_io & (k // _C)) == 0)
            arrs = exchange(arrs, j, is_lo == up, r_io, c_io)
            j //= 2
        k *= 2

    w = _R
    while w > _KR:
        pa = [jnp.roll(a, -_KR, axis=0) for a in arrs]
        sb = (arrs[0] > pa[0]) | ((arrs[0] == pa[0]) & (arrs[1] < pa[1]))
        win = [jnp.where(sb, a, p) for a, p in zip(arrs, pa)]
        w //= 2
        arrs = [x.reshape(w // _KR, 2 * _KR, _C)[:, 0:_KR]
                .reshape(w, _C) for x in win]
        r_io, c_io = iotas(w)
        up = ((r_io >> 3) & 1) == 0
        j = _KP // 2
        while j >= 1:
            is_lo = ((c_io & j) == 0) if j < _C else ((r_io & (j // _C)) == 0)
            arrs = exchange(arrs, j, is_lo == up, r_io, c_io)
            j //= 2

    h11 = hw_ref[0:1, 0:1]
    w11 = hw_ref[0:1, 1:2]
    x1 = jnp.clip(arrs[2], 0.0, w11)
    y1 = jnp.clip(arrs[3], 0.0, h11)
    x2 = jnp.clip(arrs[4], 0.0, w11)
    y2 = jnp.clip(arrs[5], 0.0, h11)
    probs = jax.nn.sigmoid(arrs[0])
    out_ref[0, 0] = x1
    out_ref[0, 1] = y1
    out_ref[0, 2] = x2
    out_ref[0, 3] = y2
    out_ref[0, 4] = probs


def _nms_kernel(row_ref, col_ref, out_ref, m_ref):
    row = row_ref[0]
    x1r, y1r = row[0:1, :], row[1:2, :]
    x2r, y2r = row[2:3, :], row[3:4, :]
    pr = row[4:5, :]
    ws = x2r - x1r
    hs = y2r - y1r
    area_r = ws * hs
    valid = ((ws >= _MIN_SIZE) & (hs >= _MIN_SIZE)
             & (pr >= _SCORE_THRESH)).astype(jnp.float32)

    chunk = 128
    for ch in range(_KP // chunk):
        colc = col_ref[0, ch * chunk:(ch + 1) * chunk, :]
        x1c, y1c = colc[:, 0:1], colc[:, 1:2]
        x2c, y2c = colc[:, 2:3], colc[:, 3:4]
        area_c = (x2c - x1c) * (y2c - y1c)
        xx1 = jnp.maximum(x1c, x1r)
        yy1 = jnp.maximum(y1c, y1r)
        xx2 = jnp.minimum(x2c, x2r)
        yy2 = jnp.minimum(y2c, y2r)
        inter = (jnp.clip(xx2 - xx1, 0.0, None)
                 * jnp.clip(yy2 - yy1, 0.0, None))
        union = area_c + area_r - inter
        iou = inter / jnp.maximum(union, 1e-9)
        jio = jax.lax.broadcasted_iota(jnp.int32, (chunk, _KP), 0) + ch * chunk
        iio = jax.lax.broadcasted_iota(jnp.int32, (chunk, _KP), 1)
        m_ref[ch * chunk:(ch + 1) * chunk, :] = jnp.where(
            (iou > _NMS_THRESH) & (jio < iio), 1.0, 0.0)

    def cond(carry):
        return carry[1]

    def body(carry):
        keep, _ = carry
        supp = jax.lax.dot_general(
            keep, m_ref[...], (((1,), (0,)), ((), ())),
            preferred_element_type=jnp.float32)
        new = jnp.where(supp > 0.0, 0.0, valid)
        return new, jnp.any(new != keep)

    keep, _ = jax.lax.while_loop(cond, body, (valid, jnp.bool_(True)))

    out_ref[0, 0:1, :] = x1r * keep
    out_ref[0, 1:2, :] = y1r * keep
    out_ref[0, 2:3, :] = x2r * keep
    out_ref[0, 3:4, :] = y2r * keep
    out_ref[0, 4:5, :] = pr * keep


def kernel(proposals, objectness, image_height, image_width):
    B = proposals.shape[0]
    f32 = jnp.float32

    scores = jnp.concatenate(
        [objectness.astype(f32),
         jnp.full((B, _NS - _N), _NEG_INF, f32)], axis=1).reshape(B, _R, _C)
    boxes = jnp.concatenate(
        [jnp.transpose(proposals.astype(f32), (0, 2, 1)),
         jnp.zeros((B, 4, _NS - _N), f32)], axis=2).reshape(B, 4, _R, _C)
    hw = jnp.stack([jnp.asarray(image_height, f32),
                    jnp.asarray(image_width, f32)]).reshape(1, 2)

    top = pl.pallas_call(
        _topk_kernel,
        grid=(B,),
        in_specs=[
            pl.BlockSpec((1, _R, _C), lambda b: (b, 0, 0)),
            pl.BlockSpec((1, 4, _R, _C), lambda b: (b, 0, 0, 0)),
            pl.BlockSpec((1, 2), lambda b: (0, 0)),
        ],
        out_specs=pl.BlockSpec((1, 5, _KR, _C), lambda b: (b, 0, 0, 0)),
        out_shape=jax.ShapeDtypeStruct((B, 5, _KR, _C), f32),
    )(scores, boxes, hw)

    rowdat = top.reshape(B, 5, _KP)
    coldat = jnp.transpose(rowdat, (0, 2, 1))

    out = pl.pallas_call(
        _nms_kernel,
        grid=(B,),
        in_specs=[
            pl.BlockSpec((1, 5, _KP), lambda b: (b, 0, 0)),
            pl.BlockSpec((1, _KP, 5), lambda b: (b, 0, 0)),
        ],
        out_specs=pl.BlockSpec((1, 5, _KP), lambda b: (b, 0, 0)),
        out_shape=jax.ShapeDtypeStruct((B, 5, _KP), f32),
        scratch_shapes=[pltpu.VMEM((_KP, _KP), f32)],
    )(rowdat, coldat)

    return jnp.transpose(out, (0, 2, 1))[:, :_K, :]
